# Initial kernel scaffold; baseline (speedup 1.0000x reference)
import jax
import jax.numpy as jnp
from jax.experimental import pallas as pl


def kernel(edge_index, edge_attrs, emb_weight):
    raise NotImplementedError("write your pallas kernel here")


# speedup vs baseline: 16.9448x; 16.9448x over previous
import jax
import jax.numpy as jnp
from jax import lax
from jax.experimental import pallas as pl
from jax.experimental.pallas import tpu as pltpu
from jax.experimental.pallas import tpu_sc as plsc

N_REAL = 100000
N_PAD = 100096
DUMMY = N_REAL
E_REAL = 1600000
E_PAD = 1638400
NCHUNK = 12800
H = 16
RB = 256
NDC = 391
NTILES = 16
STRIPE = 6256


def _body(src_hbm, dst_hbm, emb_hbm, out_hbm, z_hbm,
          A_sh, dis_sh,
          sidx, didx, rowb, atile, btile, dtile, z1d, ones,
          gsem, ssem):
    cid = lax.axis_index("c")
    sid = lax.axis_index("s")

    def _fz1(v, c):
        z1d[pl.ds(v * 16, 16)] = jnp.zeros((16,), jnp.float32)
        return c
    lax.fori_loop(0, 98, _fz1, 0)

    def _fo(v, c):
        ones[pl.ds(v * 16, 16)] = jnp.full((16,), 1.0, jnp.float32)
        return c
    lax.fori_loop(0, 8, _fo, 0)

    dstripe = sid * STRIPE
    for k in range(3):
        pltpu.sync_copy(z1d.at[pl.ds(0, 1568)],
                        dis_sh.at[pl.ds(dstripe + k * 1568, 1568)])
    pltpu.sync_copy(z1d.at[pl.ds(0, 1552)],
                    dis_sh.at[pl.ds(dstripe + 4704, 1552)])
    plsc.subcore_barrier()

    ebase = sid * 800
    def _deg_stage(st, c):
        pltpu.sync_copy(dst_hbm.at[pl.ds(ebase + st * 16, 16)], didx)
        hs = []
        for j in range(16):
            hs.append(pltpu.async_copy(ones, dis_sh.at[didx.at[j]], ssem,
                                       add=True))
        for h in hs:
            h.wait()
        return c
    lax.fori_loop(0, 50, _deg_stage, 0)
    plsc.subcore_barrier()

    nck = 24 + jnp.where(sid < NDC - 24 * NTILES, 1, 0)

    def _rsq_chunk(k, c):
        r0 = (sid + k * 16) * RB
        pltpu.sync_copy(dis_sh.at[pl.ds(r0, RB)], dtile)
        def _rsq(v, cc):
            x = dtile[pl.ds(v * 16, 16)]
            xi = lax.bitcast_convert_type(x, jnp.int32)
            hh = jnp.int32(0x5F3759DF) - (xi >> 1)
            y = lax.bitcast_convert_type(hh, jnp.float32)
            for _ in range(4):
                y = y * (1.5 - 0.5 * x * y * y)
            y = jnp.where(x > 0.5, y, 0.0)
            dtile[pl.ds(v * 16, 16)] = y
            return cc
        lax.fori_loop(0, RB // 16, _rsq, 0)
        pltpu.sync_copy(dtile, dis_sh.at[pl.ds(r0, RB)])
        return c
    lax.fori_loop(0, nck, _rsq_chunk, 0)
    plsc.subcore_barrier()

    def _init_chunk(k, c):
        r0 = (sid + k * 16) * RB
        pltpu.sync_copy(emb_hbm.at[cid, pl.ds(r0, RB)], btile)
        pltpu.sync_copy(dis_sh.at[pl.ds(r0, RB)], dtile)
        def _row(i, cc):
            sp = plsc.load_gather(dtile, [jnp.full((16,), i, jnp.int32)])
            e = btile[i]
            atile[i] = sp * e
            btile[i] = 0.25 * e
            return cc
        lax.fori_loop(0, RB, _row, 0)
        pltpu.sync_copy(atile, z_hbm.at[cid, pl.ds(r0, RB)])
        pltpu.sync_copy(btile, out_hbm.at[cid, pl.ds(r0, RB)])
        return c
    lax.fori_loop(0, nck, _init_chunk, 0)
    plsc.subcore_barrier()

    zc = z_hbm.at[cid]

    for layer in (1, 2, 3):
        last = layer == 3

        def _zb(i, c):
            btile[i] = jnp.zeros((16,), jnp.float32)
            return c
        lax.fori_loop(0, RB, _zb, 0)
        for k in range(24):
            pltpu.sync_copy(btile, A_sh.at[pl.ds(sid * STRIPE + k * RB, RB)])
        pltpu.sync_copy(btile.at[pl.ds(0, 112)],
                        A_sh.at[pl.ds(sid * STRIPE + 24 * RB, 112)])
        plsc.subcore_barrier()

        def _stage(st, c):
            pltpu.sync_copy(src_hbm.at[pl.ds(ebase + st * 16, 16)], sidx)
            pltpu.sync_copy(dst_hbm.at[pl.ds(ebase + st * 16, 16)], didx)
            def _grp(jj, cc):
                j0 = jj * 4
                gs = []
                for b in range(4):
                    gs.append(pltpu.async_copy(zc.at[sidx.at[j0 + b]],
                                               rowb.at[b], gsem))
                for h in gs:
                    h.wait()
                ss = []
                for b in range(4):
                    ss.append(pltpu.async_copy(rowb.at[b],
                                               A_sh.at[didx.at[j0 + b]],
                                               ssem, add=True))
                for h in ss:
                    h.wait()
                return cc
            lax.fori_loop(0, 4, _grp, 0)
            return c
        lax.fori_loop(0, 50, _stage, 0)
        plsc.subcore_barrier()

        def _dchunk(k, c):
            r0 = (sid + k * 16) * RB
            pltpu.sync_copy(A_sh.at[pl.ds(r0, RB)], atile)
            pltpu.sync_copy(dis_sh.at[pl.ds(r0, RB)], dtile)
            pltpu.sync_copy(out_hbm.at[cid, pl.ds(r0, RB)], btile)
            def _row(i, cc):
                sp = plsc.load_gather(dtile, [jnp.full((16,), i, jnp.int32)])
                cur = sp * atile[i]
                btile[i] = btile[i] + 0.25 * cur
                if not last:
                    atile[i] = sp * cur
                return cc
            lax.fori_loop(0, RB, _row, 0)
            pltpu.sync_copy(btile, out_hbm.at[cid, pl.ds(r0, RB)])
            if not last:
                pltpu.sync_copy(atile, z_hbm.at[cid, pl.ds(r0, RB)])
            return c
        lax.fori_loop(0, nck, _dchunk, 0)
        plsc.subcore_barrier()


@jax.jit
def _gnn(src2, dst2, embs):
    mesh = plsc.VectorSubcoreMesh(core_axis_name="c", subcore_axis_name="s")
    f = pl.kernel(
        _body,
        out_type=(
            jax.ShapeDtypeStruct((2, N_PAD, H), jnp.float32),
            jax.ShapeDtypeStruct((2, N_PAD, H), jnp.float32),
        ),
        mesh=mesh,
        compiler_params=pltpu.CompilerParams(
            needs_layout_passes=False, use_tc_tiling_on_sc=False),
        scratch_types=(
            pltpu.VMEM_SHARED((N_PAD, H), jnp.float32),
            pltpu.VMEM_SHARED((N_PAD,), jnp.float32),
            pltpu.VMEM((16, 128), jnp.int32),
            pltpu.VMEM((16, 128), jnp.int32),
            pltpu.VMEM((4, 128, H), jnp.float32),
            pltpu.VMEM((RB, H), jnp.float32),
            pltpu.VMEM((RB, H), jnp.float32),
            pltpu.VMEM((RB,), jnp.float32),
            pltpu.VMEM((1568,), jnp.float32),
            pltpu.VMEM((128,), jnp.float32),
            pltpu.SemaphoreType.DMA,
            pltpu.SemaphoreType.DMA,
        ),
    )
    return f(src2, dst2, embs)


def kernel(edge_index, edge_attrs, emb_weight):
    del edge_attrs
    src = edge_index[0]
    dst = edge_index[1]
    pad = jnp.full((E_PAD - E_REAL,), DUMMY, jnp.int32)
    src2 = jnp.concatenate([src, pad]).reshape(NCHUNK, 128)
    dst2 = jnp.concatenate([dst, pad]).reshape(NCHUNK, 128)
    embp = jnp.pad(emb_weight, ((0, N_PAD - N_REAL), (0, 0)))
    embs = jnp.stack([embp[:, :H], embp[:, H:]])
    out2, _ = _gnn(src2, dst2, embs)
    out = jnp.concatenate([out2[0, :N_REAL], out2[1, :N_REAL]], axis=1)
    return (emb_weight, out)


# speedup vs baseline: 17.5332x; 1.0347x over previous
import jax
import jax.numpy as jnp
from jax import lax
from jax.experimental import pallas as pl
from jax.experimental.pallas import tpu as pltpu
from jax.experimental.pallas import tpu_sc as plsc

N_REAL = 100000
N_PAD = 100096
DUMMY = N_REAL
E_REAL = 1600000
E_PAD = 1638400
NCHUNK = 12800
H = 16
RB = 128
NDC = 782
NTILES = 16
STRIPE = 6256


def _body(src_hbm, dst_hbm, emb_hbm, out_hbm, z_hbm,
          A_sh, dis_sh,
          sidx, didx, rowb, atile, btile, dtile,
          gsem, ssem, ssem2):
    cid = lax.axis_index("c")
    sid = lax.axis_index("s")

    def _fz1(v, c):
        dtile[pl.ds(v * 16, 16)] = jnp.zeros((16,), jnp.float32)
        return c
    lax.fori_loop(0, 8, _fz1, 0)

    dstripe = sid * STRIPE
    def _zdeg(k, c):
        pltpu.sync_copy(dtile, dis_sh.at[pl.ds(dstripe + k * 128, 128)])
        return c
    lax.fori_loop(0, 48, _zdeg, 0)
    pltpu.sync_copy(dtile.at[pl.ds(0, 112)],
                    dis_sh.at[pl.ds(dstripe + 6144, 112)])
    plsc.subcore_barrier()

    def _fo(v, c):
        dtile[pl.ds(v * 16, 16)] = jnp.full((16,), 1.0, jnp.float32)
        return c
    lax.fori_loop(0, 8, _fo, 0)

    ebase = sid * 800
    def _deg_stage(st, c):
        pltpu.sync_copy(dst_hbm.at[pl.ds(ebase + st * 16, 16)], didx)
        hs = []
        for j in range(16):
            hs.append(pltpu.async_copy(dtile.at[pl.ds(0, 128)],
                                       dis_sh.at[didx.at[j]], ssem,
                                       add=True))
        for h in hs:
            h.wait()
        return c
    lax.fori_loop(0, 50, _deg_stage, 0)
    plsc.subcore_barrier()

    nck = 48 + jnp.where(sid < NDC - 48 * NTILES, 1, 0)

    def _rsq_chunk(k, c):
        r0 = (sid + k * 16) * RB
        pltpu.sync_copy(dis_sh.at[pl.ds(r0, RB)], dtile)
        def _rsq(v, cc):
            x = dtile[pl.ds(v * 16, 16)]
            xi = lax.bitcast_convert_type(x, jnp.int32)
            hh = jnp.int32(0x5F3759DF) - (xi >> 1)
            y = lax.bitcast_convert_type(hh, jnp.float32)
            for _ in range(4):
                y = y * (1.5 - 0.5 * x * y * y)
            y = jnp.where(x > 0.5, y, 0.0)
            dtile[pl.ds(v * 16, 16)] = y
            return cc
        lax.fori_loop(0, RB // 16, _rsq, 0)
        pltpu.sync_copy(dtile, dis_sh.at[pl.ds(r0, RB)])
        return c
    lax.fori_loop(0, nck, _rsq_chunk, 0)
    plsc.subcore_barrier()

    def _init_chunk(k, c):
        r0 = (sid + k * 16) * RB
        pltpu.sync_copy(emb_hbm.at[cid, pl.ds(r0, RB)], btile)
        pltpu.sync_copy(dis_sh.at[pl.ds(r0, RB)], dtile)
        def _row(i, cc):
            sp = plsc.load_gather(dtile, [jnp.full((16,), i, jnp.int32)])
            e = btile[i]
            atile[i] = sp * e
            btile[i] = 0.25 * e
            return cc
        lax.fori_loop(0, RB, _row, 0)
        pltpu.sync_copy(atile, z_hbm.at[cid, pl.ds(r0, RB)])
        pltpu.sync_copy(btile, out_hbm.at[cid, pl.ds(r0, RB)])
        return c
    lax.fori_loop(0, nck, _init_chunk, 0)
    plsc.subcore_barrier()

    zc = z_hbm.at[cid]

    for layer in (1, 2, 3):
        last = layer == 3

        def _zb(i, c):
            btile[i] = jnp.zeros((16,), jnp.float32)
            return c
        lax.fori_loop(0, RB, _zb, 0)
        def _za(k, c):
            pltpu.sync_copy(btile, A_sh.at[pl.ds(sid * STRIPE + k * RB, RB)])
            return c
        lax.fori_loop(0, 48, _za, 0)
        pltpu.sync_copy(btile.at[pl.ds(0, 112)],
                        A_sh.at[pl.ds(sid * STRIPE + 48 * RB, 112)])
        plsc.subcore_barrier()

        def _stage(st, c):
            pltpu.sync_copy(src_hbm.at[pl.ds(ebase + st * 16, 16)], sidx)
            pltpu.sync_copy(dst_hbm.at[pl.ds(ebase + st * 16, 16)], didx)

            def fire_g(g):
                bk = (g % 2) * 4
                return [pltpu.async_copy(zc.at[sidx.at[g * 4 + b]],
                                         rowb.at[bk + b], gsem)
                        for b in range(4)]

            def fire_s(g):
                bk = (g % 2) * 4
                sem = ssem if g % 2 == 0 else ssem2
                return [pltpu.async_copy(rowb.at[bk + b],
                                         A_sh.at[didx.at[g * 4 + b]],
                                         sem, add=True)
                        for b in range(4)]

            gh = fire_g(0)
            sh_prev = None
            for g in range(4):
                for h in gh:
                    h.wait()
                sh = fire_s(g)
                if sh_prev is not None:
                    for h in sh_prev:
                        h.wait()
                if g < 3:
                    gh = fire_g(g + 1)
                sh_prev = sh
            for h in sh_prev:
                h.wait()
            return c
        lax.fori_loop(0, 50, _stage, 0)
        plsc.subcore_barrier()

        def _dchunk(k, c):
            r0 = (sid + k * 16) * RB
            pltpu.sync_copy(A_sh.at[pl.ds(r0, RB)], atile)
            pltpu.sync_copy(dis_sh.at[pl.ds(r0, RB)], dtile)
            pltpu.sync_copy(out_hbm.at[cid, pl.ds(r0, RB)], btile)
            def _row(i, cc):
                sp = plsc.load_gather(dtile, [jnp.full((16,), i, jnp.int32)])
                cur = sp * atile[i]
                btile[i] = btile[i] + 0.25 * cur
                if not last:
                    atile[i] = sp * cur
                return cc
            lax.fori_loop(0, RB, _row, 0)
            pltpu.sync_copy(btile, out_hbm.at[cid, pl.ds(r0, RB)])
            if not last:
                pltpu.sync_copy(atile, z_hbm.at[cid, pl.ds(r0, RB)])
            return c
        lax.fori_loop(0, nck, _dchunk, 0)
        plsc.subcore_barrier()


@jax.jit
def _gnn(src2, dst2, embs):
    mesh = plsc.VectorSubcoreMesh(core_axis_name="c", subcore_axis_name="s")
    f = pl.kernel(
        _body,
        out_type=(
            jax.ShapeDtypeStruct((2, N_PAD, H), jnp.float32),
            jax.ShapeDtypeStruct((2, N_PAD, H), jnp.float32),
        ),
        mesh=mesh,
        compiler_params=pltpu.CompilerParams(
            needs_layout_passes=False, use_tc_tiling_on_sc=False),
        scratch_types=(
            pltpu.VMEM_SHARED((N_PAD, H), jnp.float32),
            pltpu.VMEM_SHARED((N_PAD,), jnp.float32),
            pltpu.VMEM((16, 128), jnp.int32),
            pltpu.VMEM((16, 128), jnp.int32),
            pltpu.VMEM((8, 128, H), jnp.float32),
            pltpu.VMEM((RB, H), jnp.float32),
            pltpu.VMEM((RB, H), jnp.float32),
            pltpu.VMEM((RB,), jnp.float32),
            pltpu.SemaphoreType.DMA,
            pltpu.SemaphoreType.DMA,
            pltpu.SemaphoreType.DMA,
        ),
    )
    return f(src2, dst2, embs)


def kernel(edge_index, edge_attrs, emb_weight):
    del edge_attrs
    src = edge_index[0]
    dst = edge_index[1]
    pad = jnp.full((E_PAD - E_REAL,), DUMMY, jnp.int32)
    src2 = jnp.concatenate([src, pad]).reshape(NCHUNK, 128)
    dst2 = jnp.concatenate([dst, pad]).reshape(NCHUNK, 128)
    embp = jnp.pad(emb_weight, ((0, N_PAD - N_REAL), (0, 0)))
    embs = jnp.stack([embp[:, :H], embp[:, H:]])
    out2, _ = _gnn(src2, dst2, embs)
    out = jnp.concatenate([out2[0, :N_REAL], out2[1, :N_REAL]], axis=1)
    return (emb_weight, out)
